# initial kernel scaffold (unmeasured)
import jax
import jax.numpy as jnp
from jax import lax
from jax.experimental import pallas as pl
from jax.experimental.pallas import tpu as pltpu

N_DEV = 4


def kernel(x, w_mat):
    x = x.astype(jnp.bfloat16)
    w = w_mat.astype(jnp.bfloat16)
    m_per, k = x.shape
    _, n_per = w.shape

    def body(x_ref, w_ref, out_ref, comm_ref, send_sems, recv_sems,
             amax_comm, amax_send_sems, amax_recv_sems):
        me = lax.axis_index("i")
        left = (me + N_DEV - 1) % N_DEV
        right = (me + 1) % N_DEV

        barrier_sem = pltpu.get_barrier_semaphore()
        for nbr in (left, right):
            pl.semaphore_signal(barrier_sem, inc=1, device_id=(nbr,),
                                device_id_type=pl.DeviceIdType.MESH)
        pl.semaphore_wait(barrier_sem, 2)

        comm_ref[0] = x_ref[...]

        def gemm_store(chunk, origin):
            y = jnp.dot(chunk, w_ref[...], preferred_element_type=jnp.float32)
            y = jnp.maximum(y, 0.0)
            out_ref[pl.ds(origin * m_per, m_per), :] = y
            return jnp.max(y)

        amax = jnp.float32(0.0)
        for h in range(N_DEV - 1):
            send_slot = h % 2
            recv_slot = (h + 1) % 2
            rdma = pltpu.make_async_remote_copy(
                src_ref=comm_ref.at[send_slot],
                dst_ref=comm_ref.at[recv_slot],
                send_sem=send_sems.at[send_slot],
                recv_sem=recv_sems.at[recv_slot],
                device_id=(right,),
                device_id_type=pl.DeviceIdType.MESH,
            )
            rdma.start()
            if h == 0:
                amax = jnp.maximum(amax, gemm_store(x_ref[...], me))
            else:
                amax = jnp.maximum(
                    amax,
                    gemm_store(comm_ref[h % 2], (me + N_DEV - h) % N_DEV),
                )
            rdma.wait()
        amax = jnp.maximum(
            amax,
            gemm_store(comm_ref[(N_DEV - 1) % 2], (me + 1) % N_DEV),
        )

        amax_comm[0] = jnp.full((8, 128), amax, jnp.float32)
        g = amax
        for h in range(N_DEV - 1):
            s, r = h % 2, (h + 1) % 2
            rdma = pltpu.make_async_remote_copy(
                src_ref=amax_comm.at[s],
                dst_ref=amax_comm.at[r],
                send_sem=amax_send_sems.at[s],
                recv_sem=amax_recv_sems.at[r],
                device_id=(right,),
                device_id_type=pl.DeviceIdType.MESH,
            )
            rdma.start()
            rdma.wait()
            g = jnp.maximum(g, amax_comm[r, 0, 0])

        scale = g / 127.0
        inv = 127.0 / g
        q = jnp.clip(jnp.round(out_ref[...] * inv), 0.0, 127.0)
        out_ref[...] = q * scale

    return pl.pallas_call(
        body,
        out_shape=jax.ShapeDtypeStruct((N_DEV * m_per, n_per), jnp.float32),
        in_specs=[
            pl.BlockSpec(memory_space=pltpu.VMEM),
            pl.BlockSpec(memory_space=pltpu.VMEM),
        ],
        out_specs=pl.BlockSpec(memory_space=pltpu.VMEM),
        scratch_shapes=[
            pltpu.VMEM((2, m_per, k), jnp.bfloat16),
            pltpu.SemaphoreType.DMA((2,)),
            pltpu.SemaphoreType.DMA((2,)),
            pltpu.VMEM((2, 8, 128), jnp.float32),
            pltpu.SemaphoreType.DMA((2,)),
            pltpu.SemaphoreType.DMA((2,)),
        ],
        compiler_params=pltpu.CompilerParams(collective_id=0),
    )(x, w)


# baseline (device time: 220336 ns/iter reference)
import jax
import jax.numpy as jnp
from jax import lax
from jax.experimental import pallas as pl
from jax.experimental.pallas import tpu as pltpu

N_DEV = 4
QBLK = 512


def _gather_gemm(x, w):
    m_per, k = x.shape
    _, n_per = w.shape
    m_tot = N_DEV * m_per

    def body(x_ref, w_ref, y_ref, gmax_ref, commR, sendRs, recvRs,
             commL, sendLs, recvLs,
             amax_comm, amax_send_sems, amax_recv_sems, xsems,
             sub_send, sub_recv):
        me = lax.axis_index("i")
        left = (me + N_DEV - 1) % N_DEV
        right = (me + 1) % N_DEV

        half = m_per // 2
        xcpR = pltpu.make_async_copy(x_ref.at[pl.ds(0, half), :],
                                     commR.at[0], xsems.at[0])
        xcpL = pltpu.make_async_copy(x_ref.at[pl.ds(half, half), :],
                                     commL.at[0], xsems.at[1])
        xcpR.start()
        xcpL.start()

        barrier_sem = pltpu.get_barrier_semaphore()
        for nbr in (left, right):
            pl.semaphore_signal(barrier_sem, inc=1, device_id=(nbr,),
                                device_id_type=pl.DeviceIdType.MESH)
        pl.semaphore_wait(barrier_sem, 2)
        xcpR.wait()
        xcpL.wait()

        state = {"amax": jnp.float32(0.0)}

        def do_half(buf_ref, origin, top):
            y = jnp.dot(buf_ref[...], w_ref[...],
                        preferred_element_type=jnp.float32)
            y = jnp.maximum(y, 0.0)
            state["amax"] = jnp.maximum(state["amax"], jnp.max(y))
            row0 = origin * m_per + (0 if top else half)
            y_ref[pl.ds(row0, half), :] = y.astype(jnp.bfloat16)

        for h in range(N_DEV - 2):
            rdmaR = pltpu.make_async_remote_copy(
                src_ref=commR.at[h % 2],
                dst_ref=commR.at[(h + 1) % 2],
                send_sem=sendRs.at[h % 2],
                recv_sem=recvRs.at[(h + 1) % 2],
                device_id=(right,),
                device_id_type=pl.DeviceIdType.MESH,
            )
            rdmaL = pltpu.make_async_remote_copy(
                src_ref=commL.at[h % 2],
                dst_ref=commL.at[(h + 1) % 2],
                send_sem=sendLs.at[h % 2],
                recv_sem=recvLs.at[(h + 1) % 2],
                device_id=(left,),
                device_id_type=pl.DeviceIdType.MESH,
            )
            rdmaR.start()
            rdmaL.start()
            do_half(commR.at[h % 2], (me + N_DEV - h) % N_DEV, True)
            do_half(commL.at[h % 2], (me + h) % N_DEV, False)
            rdmaR.wait()
            rdmaL.wait()
        q = half // 2
        subs = []
        for s in range(2):
            for d, (comm, dev, off) in enumerate(
                    [(commR, right, 0), (commL, left, 2)]):
                rd = pltpu.make_async_remote_copy(
                    src_ref=comm.at[0, pl.ds(s * q, q), :],
                    dst_ref=comm.at[1, pl.ds(s * q, q), :],
                    send_sem=sub_send.at[off + s],
                    recv_sem=sub_recv.at[off + s],
                    device_id=(dev,),
                    device_id_type=pl.DeviceIdType.MESH,
                )
                rd.start()
                subs.append(rd)
        do_half(commR.at[0], (me + 2) % N_DEV, True)
        do_half(commL.at[0], (me + 2) % N_DEV, False)

        def do_quarter(buf_ref, origin, top, s):
            y = jnp.dot(buf_ref[...], w_ref[...],
                        preferred_element_type=jnp.float32)
            y = jnp.maximum(y, 0.0)
            state["amax"] = jnp.maximum(state["amax"], jnp.max(y))
            row0 = origin * m_per + (0 if top else half) + s * q
            y_ref[pl.ds(row0, q), :] = y.astype(jnp.bfloat16)

        for s in range(2):
            subs[2 * s].wait_recv()
            do_quarter(commR.at[1, pl.ds(s * q, q), :], (me + 1) % N_DEV,
                       True, s)
            subs[2 * s + 1].wait_recv()
            do_quarter(commL.at[1, pl.ds(s * q, q), :],
                       (me + N_DEV - 1) % N_DEV, False, s)
        for rd in subs:
            rd.wait_send()

        amax_comm[0] = jnp.full((8, 128), state["amax"], jnp.float32)
        gmax = state["amax"]
        for h in range(N_DEV - 1):
            rdma = pltpu.make_async_remote_copy(
                src_ref=amax_comm.at[h % 2],
                dst_ref=amax_comm.at[(h + 1) % 2],
                send_sem=amax_send_sems.at[h % 2],
                recv_sem=amax_recv_sems.at[(h + 1) % 2],
                device_id=(right,),
                device_id_type=pl.DeviceIdType.MESH,
            )
            rdma.start()
            rdma.wait()
            gmax = jnp.maximum(gmax, amax_comm[(h + 1) % 2, 0, 0])
        gmax_ref[...] = jnp.full((8, 128), gmax, jnp.float32)

    return pl.pallas_call(
        body,
        out_shape=[
            jax.ShapeDtypeStruct((m_tot, n_per), jnp.bfloat16),
            jax.ShapeDtypeStruct((8, 128), jnp.float32),
        ],
        in_specs=[
            pl.BlockSpec(memory_space=pl.ANY),
            pl.BlockSpec(memory_space=pltpu.VMEM),
        ],
        out_specs=[
            pl.BlockSpec(memory_space=pltpu.VMEM),
            pl.BlockSpec(memory_space=pltpu.VMEM),
        ],
        scratch_shapes=[
            pltpu.VMEM((2, m_per // 2, k), jnp.bfloat16),
            pltpu.SemaphoreType.DMA((2,)),
            pltpu.SemaphoreType.DMA((2,)),
            pltpu.VMEM((2, m_per // 2, k), jnp.bfloat16),
            pltpu.SemaphoreType.DMA((2,)),
            pltpu.SemaphoreType.DMA((2,)),
            pltpu.VMEM((2, 8, 128), jnp.float32),
            pltpu.SemaphoreType.DMA((2,)),
            pltpu.SemaphoreType.DMA((2,)),
            pltpu.SemaphoreType.DMA((2,)),
            pltpu.SemaphoreType.DMA((4,)),
            pltpu.SemaphoreType.DMA((4,)),
        ],
        compiler_params=pltpu.CompilerParams(
            collective_id=0,
            vmem_limit_bytes=56 * 1024 * 1024,
        ),
    )(x, w)


def _quantize(y, gmax):
    m_tot, n_per = y.shape

    def body(y_ref, g_ref, o_ref):
        g = g_ref[0, 0]
        scale = g / 127.0
        inv = 127.0 / g
        v = y_ref[...].astype(jnp.float32)
        o_ref[...] = jnp.clip(jnp.round(v * inv), 0.0, 127.0) * scale

    return pl.pallas_call(
        body,
        grid=(m_tot // QBLK,),
        in_specs=[
            pl.BlockSpec((QBLK, n_per), lambda i: (i, 0)),
            pl.BlockSpec((8, 128), lambda i: (0, 0)),
        ],
        out_specs=pl.BlockSpec((QBLK, n_per), lambda i: (i, 0)),
        out_shape=jax.ShapeDtypeStruct((m_tot, n_per), jnp.float32),
    )(y, gmax)


def kernel(x, w_mat):
    x = x.astype(jnp.bfloat16)
    w = w_mat.astype(jnp.bfloat16)
    y, gmax = _gather_gemm(x, w)
    return _quantize(y, gmax)
